# baseline (speedup 1.0000x reference)
import functools

import jax
import jax.numpy as jnp
import numpy as np
from jax import lax
from jax.experimental import pallas as pl
from jax.experimental.pallas import tpu as pltpu
from jax.experimental.pallas import tpu_sc as plsc

NUM_POS_FEATS = 32
FRAME_RANGE = 5
TEMPERATURE = 10000.0

NC = 2
NS = 16
L = 16
D = FRAME_RANGE * 4


def _resolve_rows_sc(tr1d, det1d, r_idx, rd_idx, g_idx, M):
    B = g_idx.shape[0]
    OWN = M // NS
    HALF = B // NC
    CHUNK = HALF // NS
    NQ = CHUNK // 128

    mesh = plsc.VectorSubcoreMesh(core_axis_name="c", subcore_axis_name="s")

    @functools.partial(
        pl.kernel,
        mesh=mesh,
        compiler_params=pltpu.CompilerParams(needs_layout_passes=False,
                                             use_tc_tiling_on_sc=False),
        out_type=jax.ShapeDtypeStruct((B * D,), jnp.float32),
        scratch_types=[
            pltpu.VMEM((OWN,), jnp.int32),
            pltpu.VMEM((B,), jnp.int32),
            pltpu.VMEM((HALF,), jnp.int32),
            pltpu.VMEM((HALF,), jnp.int32),
            pltpu.VMEM((CHUNK,), jnp.int32),
            pltpu.VMEM((CHUNK,), jnp.int32),
            pltpu.VMEM((D * NQ, 128), jnp.int32),
            pltpu.VMEM((NQ, 128), jnp.int32),
            pltpu.VMEM((CHUNK,), jnp.int32),
            pltpu.VMEM((4 * NQ, 128), jnp.int32),
            pltpu.VMEM((CHUNK * 4,), jnp.float32),
            pltpu.VMEM((CHUNK * D,), jnp.float32),
            pltpu.VMEM_SHARED((NS, HALF), jnp.int32),
            pltpu.SemaphoreType.DMA,
            pltpu.SemaphoreType.DMA,
        ],
    )
    def k(tr_hbm, det_hbm, r_hbm, rd_hbm, g_hbm, out_hbm,
          marker, r_all, g_sc, wpart, tmpc, wchunk, tidx, ridx, rdvals,
          didx, detflat, rows, stage, sem0, sem1):
        c = lax.axis_index("c")
        s = lax.axis_index("s")
        own_base = s * OWN
        chunk_off = s * CHUNK
        gstart = c * HALF + s * CHUNK

        iota = lax.iota(jnp.int32, L)
        zeros = jnp.zeros((L,), jnp.int32)

        pltpu.sync_copy(g_hbm.at[pl.ds(c * HALF, HALF)], g_sc)
        pltpu.sync_copy(r_hbm, r_all)

        def owned(a16):
            local = a16 - own_base
            mask = (local >= 0) & (local < OWN)
            lcl = jnp.clip(local, 0, OWN - 1)
            return lcl, mask

        def ph_a(i, _):
            lcl, mask = owned(g_sc[pl.ds(i * L, L)])
            plsc.store_scatter(marker, [lcl], zeros, mask=mask)
            return 0
        lax.fori_loop(0, HALF // L, ph_a, 0)

        def ph_b(j, _):
            lcl, mask = owned(r_all[pl.ds(j * L, L)])
            plsc.store_scatter(marker, [lcl], j * L + iota + 1, mask=mask)
            return 0
        lax.fori_loop(0, B // L, ph_b, 0)

        def ph_c(i, _):
            lcl, mask = owned(g_sc[pl.ds(i * L, L)])
            w16 = plsc.load_gather(marker, [lcl], mask=mask)
            wpart[pl.ds(i * L, L)] = jnp.where(mask, w16, 0)
            return 0
        lax.fori_loop(0, HALF // L, ph_c, 0)

        pltpu.sync_copy(wpart, stage.at[s])
        plsc.subcore_barrier()

        def zerow(i, _):
            wchunk[pl.ds(i * L, L)] = zeros
            return 0
        lax.fori_loop(0, CHUNK // L, zerow, 0)
        for t in range(NS):
            pltpu.sync_copy(stage.at[t, pl.ds(chunk_off, CHUNK)], tmpc)

            def mx(i, _):
                wchunk[pl.ds(i * L, L)] = jnp.maximum(
                    wchunk[pl.ds(i * L, L)], tmpc[pl.ds(i * L, L)])
                return 0
            lax.fori_loop(0, CHUNK // L, mx, 0)

        def comb(i, _):
            acc = wchunk[pl.ds(i * L, L)]
            mask = acc > 0
            ii = i * L + iota
            ridx[i // 8, pl.ds((i % 8) * L, L)] = jnp.where(
                mask, acc - 1, gstart + ii)
            return 0
        lax.fori_loop(0, CHUNK // L, comb, 0)

        def build_tidx(i, _):
            g20 = g_sc[pl.ds(chunk_off + i * L, L)] * D
            q, sub = i // 8, i % 8
            for m in range(D):
                tidx[q * D + m, pl.ds(sub * L, L)] = g20 + m
            return 0
        lax.fori_loop(0, CHUNK // L, build_tidx, 0)

        for q in range(NQ):
            cps = []
            for m in range(D):
                cps.append(pltpu.async_copy(
                    tr_hbm.at[tidx.at[q * D + m]],
                    rows.at[pl.ds((q * D + m) * 128, 128)], sem0))
            cps.append(pltpu.async_copy(
                rd_hbm.at[ridx.at[q]],
                rdvals.at[pl.ds(q * 128, 128)], sem1))
            for cp in cps:
                cp.wait()


        def build_didx(i, _):
            rv4 = rdvals[pl.ds(i * L, L)] * 4
            q, sub = i // 8, i % 8
            for cc in range(4):
                didx[q * 4 + cc, pl.ds(sub * L, L)] = rv4 + cc
            return 0
        lax.fori_loop(0, CHUNK // L, build_didx, 0)

        cps = []
        for q in range(NQ):
            for cc in range(4):
                cps.append(pltpu.async_copy(
                    det_hbm.at[didx.at[q * 4 + cc]],
                    detflat.at[pl.ds((q * 4 + cc) * 128, 128)], sem0))
        for cp in cps:
            cp.wait()

        def patch3(i, _):
            mask = wchunk[pl.ds(i * L, L)] > 0
            base = (i // 8) * (D * 128) + (i % 8) * L
            dbase = (i // 8) * 512 + (i % 8) * L
            for cc in range(4):
                dv = detflat[pl.ds(dbase + cc * 128, L)]
                for f in range(FRAME_RANGE):
                    m = f * 4 + cc
                    old = rows[pl.ds(base + m * 128, L)]
                    rows[pl.ds(base + m * 128, L)] = jnp.where(mask, dv, old)
            return 0
        lax.fori_loop(0, CHUNK // L, patch3, 0)

        pltpu.sync_copy(rows, out_hbm.at[pl.ds(gstart * D, CHUNK * D)])

    return k(tr1d, det1d, r_idx, rd_idx, g_idx)


def _sine_encode_tc(rows):
    B, _ = rows.shape
    OUTD = D * NUM_POS_FEATS
    BLK = 512

    dim_t = np.arange(NUM_POS_FEATS, dtype=np.float32)
    dim_t = np.float32(TEMPERATURE) ** (2 * np.floor(dim_t / 2).astype(np.float32)
                                        / np.float32(NUM_POS_FEATS))
    w16 = (np.float32(2.0) * np.float32(np.pi)) / dim_t[0::2]
    w32 = np.concatenate([w16, w16])
    off32 = np.concatenate([np.full(16, np.pi / 2, np.float32),
                            np.zeros(16, np.float32)])
    w128 = jnp.asarray(np.tile(w32, 4), jnp.float32)[None, :]
    off128 = jnp.asarray(np.tile(off32, 4), jnp.float32)[None, :]

    def enc(w_ref, o_ref, rows_ref, out_ref):
        w = w_ref[...]
        off = o_ref[...]
        for f in range(FRAME_RANGE):
            xs = [jnp.broadcast_to(rows_ref[:, f * 4 + cc:f * 4 + cc + 1],
                                   (BLK, 32)) for cc in range(4)]
            x128 = jnp.concatenate(xs, axis=1)
            out_ref[:, f * 128:(f + 1) * 128] = jnp.sin(x128 * w + off)

    return pl.pallas_call(
        enc,
        grid=(B // BLK,),
        in_specs=[pl.BlockSpec((1, 128), lambda i: (0, 0)),
                  pl.BlockSpec((1, 128), lambda i: (0, 0)),
                  pl.BlockSpec((BLK, D), lambda i: (i, 0))],
        out_specs=pl.BlockSpec((BLK, OUTD), lambda i: (i, 0)),
        out_shape=jax.ShapeDtypeStruct((B, OUTD), jnp.float32),
    )(w128, off128, rows)


def kernel(tracklets, detections, replace_track_indices, replace_det_indices,
           tracklets_indices):
    M = tracklets.shape[0]
    B = tracklets_indices.shape[0]
    tr1d = tracklets.reshape(M * D)
    det1d = detections.reshape(B * 4)
    flat = _resolve_rows_sc(tr1d, det1d, replace_track_indices,
                            replace_det_indices, tracklets_indices, M)
    rows = flat.reshape(B // 128, D, 128).transpose(0, 2, 1).reshape(B, D)
    return _sine_encode_tc(rows)


# speedup vs baseline: 3.7210x; 3.7210x over previous
import functools

import jax
import jax.numpy as jnp
import numpy as np
from jax import lax
from jax.experimental import pallas as pl
from jax.experimental.pallas import tpu as pltpu
from jax.experimental.pallas import tpu_sc as plsc

NUM_POS_FEATS = 32
FRAME_RANGE = 5
TEMPERATURE = 10000.0

NC = 2
NS = 16
L = 16
D = FRAME_RANGE * 4


def _resolve_rows_sc(trT, detT, r_idx, rd_idx, g_idx, M):
    B = g_idx.shape[0]
    OWN = M // NS
    HALF = B // NC
    CHUNK = HALF // NS
    NQ = CHUNK // 128

    mesh = plsc.VectorSubcoreMesh(core_axis_name="c", subcore_axis_name="s")

    @functools.partial(
        pl.kernel,
        mesh=mesh,
        compiler_params=pltpu.CompilerParams(needs_layout_passes=False,
                                             use_tc_tiling_on_sc=False),
        out_type=jax.ShapeDtypeStruct((B * D,), jnp.float32),
        scratch_types=[
            pltpu.VMEM((OWN,), jnp.int32),
            pltpu.VMEM((B,), jnp.int32),
            pltpu.VMEM((HALF,), jnp.int32),
            pltpu.VMEM((HALF,), jnp.int32),
            pltpu.VMEM((CHUNK,), jnp.int32),
            pltpu.VMEM((CHUNK,), jnp.int32),
            pltpu.VMEM((NQ, 128), jnp.int32),
            pltpu.VMEM((NQ, 128), jnp.int32),
            pltpu.VMEM((NQ, 128), jnp.int32),
            pltpu.VMEM((CHUNK * 4,), jnp.float32),
            pltpu.VMEM((CHUNK * D,), jnp.float32),
            pltpu.VMEM_SHARED((NS, HALF), jnp.int32),
            pltpu.SemaphoreType.DMA,
            pltpu.SemaphoreType.DMA,
        ],
    )
    def k(tr_hbm, det_hbm, r_hbm, rd_hbm, g_hbm, out_hbm,
          marker, r_all, g_sc, wpart, tmpc, wchunk, gq, ridx, rdv2,
          detflat, rows, stage, sem0, sem1):
        c = lax.axis_index("c")
        s = lax.axis_index("s")
        own_base = s * OWN
        chunk_off = s * CHUNK
        gstart = c * HALF + s * CHUNK

        iota = lax.iota(jnp.int32, L)
        zeros = jnp.zeros((L,), jnp.int32)

        pltpu.sync_copy(g_hbm.at[pl.ds(c * HALF, HALF)], g_sc)
        pltpu.sync_copy(r_hbm, r_all)

        def owned(a16):
            local = a16 - own_base
            mask = (local >= 0) & (local < OWN)
            lcl = jnp.clip(local, 0, OWN - 1)
            return lcl, mask

        def ph_a(i, _):
            lcl, mask = owned(g_sc[pl.ds(i * L, L)])
            plsc.store_scatter(marker, [lcl], zeros, mask=mask)
            return 0
        lax.fori_loop(0, HALF // L, ph_a, 0)

        def ph_b(j, _):
            lcl, mask = owned(r_all[pl.ds(j * L, L)])
            plsc.store_scatter(marker, [lcl], j * L + iota + 1, mask=mask)
            return 0
        lax.fori_loop(0, B // L, ph_b, 0)

        def ph_c(i, _):
            lcl, mask = owned(g_sc[pl.ds(i * L, L)])
            w16 = plsc.load_gather(marker, [lcl], mask=mask)
            wpart[pl.ds(i * L, L)] = jnp.where(mask, w16, 0)
            return 0
        lax.fori_loop(0, HALF // L, ph_c, 0)

        pltpu.sync_copy(wpart, stage.at[s])
        plsc.subcore_barrier()

        def zerow(i, _):
            wchunk[pl.ds(i * L, L)] = zeros
            return 0
        lax.fori_loop(0, CHUNK // L, zerow, 0)
        for t in range(NS):
            pltpu.sync_copy(stage.at[t, pl.ds(chunk_off, CHUNK)], tmpc)

            def mx(i, _):
                wchunk[pl.ds(i * L, L)] = jnp.maximum(
                    wchunk[pl.ds(i * L, L)], tmpc[pl.ds(i * L, L)])
                return 0
            lax.fori_loop(0, CHUNK // L, mx, 0)

        def comb(i, _):
            acc = wchunk[pl.ds(i * L, L)]
            mask = acc > 0
            ii = i * L + iota
            q, sub = i // 8, (i % 8) * L
            ridx[q, pl.ds(sub, L)] = jnp.where(mask, acc - 1, gstart + ii)
            gq[q, pl.ds(sub, L)] = g_sc[pl.ds(chunk_off + i * L, L)]
            return 0
        lax.fori_loop(0, CHUNK // L, comb, 0)

        for q in range(NQ):
            cps = []
            for m in range(D):
                cps.append(pltpu.async_copy(
                    tr_hbm.at[m].at[gq.at[q]],
                    rows.at[pl.ds((q * D + m) * 128, 128)], sem0))
            cps.append(pltpu.async_copy(
                rd_hbm.at[ridx.at[q]], rdv2.at[q], sem1))
            for cp in cps:
                cp.wait()

        cps = []
        for q in range(NQ):
            for cc in range(4):
                cps.append(pltpu.async_copy(
                    det_hbm.at[cc].at[rdv2.at[q]],
                    detflat.at[pl.ds((q * 4 + cc) * 128, 128)], sem0))
        for cp in cps:
            cp.wait()

        def patch3(i, _):
            mask = wchunk[pl.ds(i * L, L)] > 0
            base = (i // 8) * (D * 128) + (i % 8) * L
            dbase = (i // 8) * 512 + (i % 8) * L
            for cc in range(4):
                dv = detflat[pl.ds(dbase + cc * 128, L)]
                for f in range(FRAME_RANGE):
                    m = f * 4 + cc
                    old = rows[pl.ds(base + m * 128, L)]
                    rows[pl.ds(base + m * 128, L)] = jnp.where(mask, dv, old)
            return 0
        lax.fori_loop(0, CHUNK // L, patch3, 0)

        pltpu.sync_copy(rows, out_hbm.at[pl.ds(gstart * D, CHUNK * D)])

    return k(trT, detT, r_idx, rd_idx, g_idx)


def _sine_encode_tc(rows):
    B, _ = rows.shape
    OUTD = D * NUM_POS_FEATS
    BLK = 512

    dim_t = np.arange(NUM_POS_FEATS, dtype=np.float32)
    dim_t = np.float32(TEMPERATURE) ** (2 * np.floor(dim_t / 2).astype(np.float32)
                                        / np.float32(NUM_POS_FEATS))
    w16 = (np.float32(2.0) * np.float32(np.pi)) / dim_t[0::2]
    w32 = np.concatenate([w16, w16])
    off32 = np.concatenate([np.full(16, np.pi / 2, np.float32),
                            np.zeros(16, np.float32)])
    w128 = jnp.asarray(np.tile(w32, 4), jnp.float32)[None, :]
    off128 = jnp.asarray(np.tile(off32, 4), jnp.float32)[None, :]

    def enc(w_ref, o_ref, rows_ref, out_ref):
        w = w_ref[...]
        off = o_ref[...]
        for f in range(FRAME_RANGE):
            xs = [jnp.broadcast_to(rows_ref[:, f * 4 + cc:f * 4 + cc + 1],
                                   (BLK, 32)) for cc in range(4)]
            x128 = jnp.concatenate(xs, axis=1)
            out_ref[:, f * 128:(f + 1) * 128] = jnp.sin(x128 * w + off)

    return pl.pallas_call(
        enc,
        grid=(B // BLK,),
        in_specs=[pl.BlockSpec((1, 128), lambda i: (0, 0)),
                  pl.BlockSpec((1, 128), lambda i: (0, 0)),
                  pl.BlockSpec((BLK, D), lambda i: (i, 0))],
        out_specs=pl.BlockSpec((BLK, OUTD), lambda i: (i, 0)),
        out_shape=jax.ShapeDtypeStruct((B, OUTD), jnp.float32),
    )(w128, off128, rows)


def kernel(tracklets, detections, replace_track_indices, replace_det_indices,
           tracklets_indices):
    M = tracklets.shape[0]
    B = tracklets_indices.shape[0]
    trT = tracklets.reshape(M, D).T
    detT = detections.T
    flat = _resolve_rows_sc(trT, detT, replace_track_indices,
                            replace_det_indices, tracklets_indices, M)
    rows = flat.reshape(B // 128, D, 128).transpose(0, 2, 1).reshape(B, D)
    return _sine_encode_tc(rows)


# speedup vs baseline: 12.1695x; 3.2705x over previous
import functools

import jax
import jax.numpy as jnp
import numpy as np
from jax import lax
from jax.experimental import pallas as pl
from jax.experimental.pallas import tpu as pltpu
from jax.experimental.pallas import tpu_sc as plsc

NUM_POS_FEATS = 32
FRAME_RANGE = 5
TEMPERATURE = 10000.0

NC = 2
NS = 16
L = 16
D = FRAME_RANGE * 4


def _resolve_rows_sc(planes, detT, r_idx, rd_idx, g_idx, M):
    B = g_idx.shape[0]
    OWN = M // NS
    HALF = B // NC
    CHUNK = HALF // NS
    NQ = CHUNK // 128

    mesh = plsc.VectorSubcoreMesh(core_axis_name="c", subcore_axis_name="s")

    @functools.partial(
        pl.kernel,
        mesh=mesh,
        compiler_params=pltpu.CompilerParams(needs_layout_passes=False,
                                             use_tc_tiling_on_sc=False),
        out_type=jax.ShapeDtypeStruct((B * D,), jnp.float32),
        scratch_types=[
            pltpu.VMEM((OWN,), jnp.int32),
            pltpu.VMEM((B,), jnp.int32),
            pltpu.VMEM((HALF,), jnp.int32),
            pltpu.VMEM((HALF,), jnp.int32),
            pltpu.VMEM((CHUNK,), jnp.int32),
            pltpu.VMEM((CHUNK,), jnp.int32),
            pltpu.VMEM((NQ, 128), jnp.int32),
            pltpu.VMEM((NQ, 128), jnp.int32),
            pltpu.VMEM((NQ, 128), jnp.int32),
            pltpu.VMEM((CHUNK * 4,), jnp.float32),
            pltpu.VMEM((CHUNK * D,), jnp.float32),
            pltpu.VMEM_SHARED((NS, HALF), jnp.int32),
            pltpu.SemaphoreType.DMA,
            pltpu.SemaphoreType.DMA,
        ],
    )
    def k(*refs):
        tr_hbm = refs[:D]
        (det_hbm, r_hbm, rd_hbm, g_hbm, out_hbm,
         marker, r_all, g_sc, wpart, tmpc, wchunk, gq, ridx, rdv2,
         detflat, rows, stage, sem0, sem1) = refs[D:]
        c = lax.axis_index("c")
        s = lax.axis_index("s")
        own_base = s * OWN
        chunk_off = s * CHUNK
        gstart = c * HALF + s * CHUNK

        iota = lax.iota(jnp.int32, L)
        zeros = jnp.zeros((L,), jnp.int32)

        pltpu.sync_copy(g_hbm.at[pl.ds(c * HALF, HALF)], g_sc)
        pltpu.sync_copy(r_hbm, r_all)

        def owned(a16):
            local = a16 - own_base
            mask = (local >= 0) & (local < OWN)
            lcl = jnp.clip(local, 0, OWN - 1)
            return lcl, mask

        def ph_a(i, _):
            lcl, mask = owned(g_sc[pl.ds(i * L, L)])
            plsc.store_scatter(marker, [lcl], zeros, mask=mask)
            return 0
        lax.fori_loop(0, HALF // L, ph_a, 0)

        def ph_b(j, _):
            lcl, mask = owned(r_all[pl.ds(j * L, L)])
            plsc.store_scatter(marker, [lcl], j * L + iota + 1, mask=mask)
            return 0
        lax.fori_loop(0, B // L, ph_b, 0)

        def ph_c(i, _):
            lcl, mask = owned(g_sc[pl.ds(i * L, L)])
            w16 = plsc.load_gather(marker, [lcl], mask=mask)
            wpart[pl.ds(i * L, L)] = jnp.where(mask, w16, 0)
            return 0
        lax.fori_loop(0, HALF // L, ph_c, 0)

        pltpu.sync_copy(wpart, stage.at[s])
        plsc.subcore_barrier()

        def zerow(i, _):
            wchunk[pl.ds(i * L, L)] = zeros
            return 0
        lax.fori_loop(0, CHUNK // L, zerow, 0)
        for t in range(NS):
            pltpu.sync_copy(stage.at[t, pl.ds(chunk_off, CHUNK)], tmpc)

            def mx(i, _):
                wchunk[pl.ds(i * L, L)] = jnp.maximum(
                    wchunk[pl.ds(i * L, L)], tmpc[pl.ds(i * L, L)])
                return 0
            lax.fori_loop(0, CHUNK // L, mx, 0)

        def comb(i, _):
            acc = wchunk[pl.ds(i * L, L)]
            mask = acc > 0
            ii = i * L + iota
            q, sub = i // 8, (i % 8) * L
            ridx[q, pl.ds(sub, L)] = jnp.where(mask, acc - 1, gstart + ii)
            gq[q, pl.ds(sub, L)] = g_sc[pl.ds(chunk_off + i * L, L)]
            return 0
        lax.fori_loop(0, CHUNK // L, comb, 0)

        for q in range(NQ):
            cps = []
            for m in range(D):
                cps.append(pltpu.async_copy(
                    tr_hbm[m].at[gq.at[q]],
                    rows.at[pl.ds((q * D + m) * 128, 128)], sem0))
            cps.append(pltpu.async_copy(
                rd_hbm.at[ridx.at[q]], rdv2.at[q], sem1))
            for cp in cps:
                cp.wait()

        cps = []
        for q in range(NQ):
            for cc in range(4):
                cps.append(pltpu.async_copy(
                    det_hbm.at[cc].at[rdv2.at[q]],
                    detflat.at[pl.ds((q * 4 + cc) * 128, 128)], sem0))
        for cp in cps:
            cp.wait()

        def patch3(i, _):
            mask = wchunk[pl.ds(i * L, L)] > 0
            base = (i // 8) * (D * 128) + (i % 8) * L
            dbase = (i // 8) * 512 + (i % 8) * L
            for cc in range(4):
                dv = detflat[pl.ds(dbase + cc * 128, L)]
                for f in range(FRAME_RANGE):
                    m = f * 4 + cc
                    old = rows[pl.ds(base + m * 128, L)]
                    rows[pl.ds(base + m * 128, L)] = jnp.where(mask, dv, old)
            return 0
        lax.fori_loop(0, CHUNK // L, patch3, 0)

        pltpu.sync_copy(rows, out_hbm.at[pl.ds(gstart * D, CHUNK * D)])

    return k(*planes, detT, r_idx, rd_idx, g_idx)


def _sine_encode_tc(rows):
    B, _ = rows.shape
    OUTD = D * NUM_POS_FEATS
    BLK = 512

    dim_t = np.arange(NUM_POS_FEATS, dtype=np.float32)
    dim_t = np.float32(TEMPERATURE) ** (2 * np.floor(dim_t / 2).astype(np.float32)
                                        / np.float32(NUM_POS_FEATS))
    w16 = (np.float32(2.0) * np.float32(np.pi)) / dim_t[0::2]
    w32 = np.concatenate([w16, w16])
    off32 = np.concatenate([np.full(16, np.pi / 2, np.float32),
                            np.zeros(16, np.float32)])
    w128 = jnp.asarray(np.tile(w32, 4), jnp.float32)[None, :]
    off128 = jnp.asarray(np.tile(off32, 4), jnp.float32)[None, :]

    def enc(w_ref, o_ref, rows_ref, out_ref):
        w = w_ref[...]
        off = o_ref[...]
        for f in range(FRAME_RANGE):
            xs = [jnp.broadcast_to(rows_ref[:, f * 4 + cc:f * 4 + cc + 1],
                                   (BLK, 32)) for cc in range(4)]
            x128 = jnp.concatenate(xs, axis=1)
            out_ref[:, f * 128:(f + 1) * 128] = jnp.sin(x128 * w + off)

    return pl.pallas_call(
        enc,
        grid=(B // BLK,),
        in_specs=[pl.BlockSpec((1, 128), lambda i: (0, 0)),
                  pl.BlockSpec((1, 128), lambda i: (0, 0)),
                  pl.BlockSpec((BLK, D), lambda i: (i, 0))],
        out_specs=pl.BlockSpec((BLK, OUTD), lambda i: (i, 0)),
        out_shape=jax.ShapeDtypeStruct((B, OUTD), jnp.float32),
    )(w128, off128, rows)


def kernel(tracklets, detections, replace_track_indices, replace_det_indices,
           tracklets_indices):
    M = tracklets.shape[0]
    B = tracklets_indices.shape[0]
    planes = [tracklets[:, f, cc] for f in range(FRAME_RANGE)
              for cc in range(4)]
    detT = detections.T
    flat = _resolve_rows_sc(planes, detT, replace_track_indices,
                            replace_det_indices, tracklets_indices, M)
    rows = flat.reshape(B // 128, D, 128).transpose(0, 2, 1).reshape(B, D)
    return _sine_encode_tc(rows)
